# baseline (speedup 1.0000x reference)
import functools

import jax
import jax.numpy as jnp
from jax import lax
from jax.experimental import pallas as pl
from jax.experimental.pallas import tpu as pltpu
from jax.experimental.pallas import tpu_sc as plsc

N = 10000
E = 320000
D = 128

NS = 16
NC = 2
CW = 80
CH = 256
BS = 8
NB = CH // BS
KB = 4
EPAD = NS * CH * CW
NPAD = 10240
ZR = NPAD // NS

DEGR = 640

RB = 2048
NBLK = 5
DB = RB // 16


def _spmm_sc(h_row, h_col, sd_r, sd_c, zeros):
    mesh = plsc.VectorSubcoreMesh(core_axis_name="c", subcore_axis_name="s")

    @functools.partial(
        pl.kernel, mesh=mesh,
        out_type=jax.ShapeDtypeStruct((NC, NPAD, D), jnp.float32),
        scratch_types=(
            [pltpu.VMEM((BS, 2, CW), jnp.int32) for _ in range(4)]
            + [pltpu.VMEM((CW, D), jnp.float32) for _ in range(KB)]
            + [pltpu.VMEM_SHARED((NPAD, D), jnp.float32)]
            + [pltpu.SemaphoreType.DMA for _ in range(4 + 2 * KB)]
        ),
    )
    def k(hr, hc, sdr, sdc, z, out, *refs):
        ibs = refs[0:4]
        rows = refs[4:4 + KB]
        agg_sh = refs[4 + KB]
        semis = refs[5 + KB:9 + KB]
        semg = refs[9 + KB:9 + 2 * KB]
        sems = refs[9 + 2 * KB:9 + 3 * KB]
        cid = lax.axis_index("c")
        sid = lax.axis_index("s")

        pltpu.sync_copy(z, agg_sh.at[pl.ds(sid * ZR, ZR)])
        plsc.subcore_barrier()

        def run(h_hbm, sd_hbm):
            def idx_issue(b, cur):
                pltpu.async_copy(
                    sd_hbm.at[sid, pl.ds(b * BS, BS)], ibs[cur], semis[cur])

            def idx_wait(cur):
                pltpu.make_async_copy(
                    sd_hbm.at[sid, pl.ds(0, BS)], ibs[cur], semis[cur]).wait()

            def gather_issue(idx_ref, p):
                pltpu.async_copy(h_hbm.at[idx_ref], rows[p], semg[p])

            def gather_wait(p):
                pltpu.make_async_copy(
                    h_hbm.at[ibs[0].at[0, 0]], rows[p], semg[p]).wait()

            def scatter_issue(idx_ref, p):
                pltpu.async_copy(rows[p], agg_sh.at[idx_ref], sems[p],
                                 add=True)

            def scatter_wait(p):
                pltpu.make_async_copy(
                    rows[p], agg_sh.at[ibs[0].at[0, 1]], sems[p]).wait()

            idx_issue(0, 0)
            idx_issue(1, 1)
            idx_issue(2, 2)
            idx_wait(0)
            gather_issue(ibs[0].at[0, 0], 0)
            gather_issue(ibs[0].at[1, 0], 1)

            def block(b, cur):
                ib_cur = ibs[cur]
                for kk in range(BS):
                    j = b * BS + kk
                    pc = kk % KB
                    pn = (kk + 2) % KB
                    pv = (kk + 3) % KB
                    if kk == 2:
                        @pl.when(b + 3 < NB)
                        def _():
                            idx_issue(b + 3, (cur + 3) % 4)
                    if kk == BS - 2:
                        @pl.when(b + 1 < NB)
                        def _():
                            idx_wait((cur + 1) % 4)
                    if kk < BS - 2:
                        nidx = ib_cur.at[kk + 2, 0]
                    else:
                        nidx = ibs[(cur + 1) % 4].at[kk - (BS - 2), 0]

                    @pl.when(j + 2 < CH)
                    def _():
                        gather_issue(nidx, pn)

                    gather_wait(pc)
                    @pl.when(j >= 1)
                    def _():
                        scatter_wait(pv)

                    scatter_issue(ib_cur.at[kk, 1], pc)

            def quad(q, carry):
                for i in range(4):
                    block(4 * q + i, i)
                return carry

            lax.fori_loop(0, NB // 4, quad, 0)
            scatter_wait((CH - 1) % KB)

        @pl.when(cid == 0)
        def _():
            run(hr, sdr)

        @pl.when(cid == 1)
        def _():
            run(hc, sdc)

        plsc.subcore_barrier()
        pltpu.sync_copy(agg_sh.at[pl.ds(sid * ZR, ZR)],
                        out.at[cid, pl.ds(sid * ZR, ZR)])

    return k(h_row, h_col, sd_r, sd_c, zeros)


def _row_scale(block2d, scales_db16):
    x3 = block2d.reshape(DB, 16, D)
    s3 = scales_db16.reshape(DB, 16, 1)
    return (x3 * s3).reshape(RB, D)


def _deg_scale(deg_block):
    return lax.rsqrt(jnp.maximum(deg_block, 1.0))


def _scale_tc(x, degs):
    def body(x_ref, d_ref, or_ref, oc_ref):
        xv = x_ref[...]
        or_ref[...] = _row_scale(xv, _deg_scale(d_ref[0, 0]))
        oc_ref[...] = _row_scale(xv, _deg_scale(d_ref[1, 0]))

    return pl.pallas_call(
        body,
        grid=(NBLK,),
        in_specs=[
            pl.BlockSpec((RB, D), lambda i: (i, 0)),
            pl.BlockSpec((2, 1, DB, 16), lambda i: (0, i, 0, 0)),
        ],
        out_specs=[
            pl.BlockSpec((RB, D), lambda i: (i, 0)),
            pl.BlockSpec((RB, D), lambda i: (i, 0)),
        ],
        out_shape=[
            jax.ShapeDtypeStruct((NBLK * RB, D), jnp.float32),
            jax.ShapeDtypeStruct((NBLK * RB, D), jnp.float32),
        ],
    )(x, degs)


def _dense_tc(agg2, parts, W_row, b_row, W_col, b_col,
              W_rs, b_rs, g_rs, be_rs, W_cs, b_cs, g_cs, be_cs,
              W_m, b_m, g_m, be_m):
    def ln(x, gamma, beta):
        mu = jnp.mean(x, axis=-1, keepdims=True)
        xc = x - mu
        var = jnp.mean(xc * xc, axis=-1, keepdims=True)
        return xc * lax.rsqrt(var + 1e-5) * gamma + beta

    def body(ar_ref, ac_ref, p_ref, wr_ref, br_ref, wc_ref, bc_ref,
             wrs_ref, brs_ref, grs_ref, bers_ref,
             wcs_ref, bcs_ref, gcs_ref, becs_ref,
             wm_ref, bm_ref, gm_ref, bem_ref, o_ref):
        ar = _row_scale(ar_ref[0], _deg_scale(p_ref[0, 0]))
        ac = _row_scale(ac_ref[0], _deg_scale(p_ref[1, 0]))
        hr = jax.nn.relu(
            jnp.dot(ar, wr_ref[...],
                    preferred_element_type=jnp.float32) + br_ref[...])
        hc = jax.nn.relu(
            jnp.dot(ac, wc_ref[...],
                    preferred_element_type=jnp.float32) + bc_ref[...])
        r = ln(jnp.dot(hr, wrs_ref[...],
                       preferred_element_type=jnp.float32) + brs_ref[...],
               grs_ref[...], bers_ref[...])
        c = ln(jnp.dot(hc, wcs_ref[...],
                       preferred_element_type=jnp.float32) + bcs_ref[...],
               gcs_ref[...], becs_ref[...])
        m = (jnp.dot(r, wm_ref[pl.ds(0, D), :],
                     preferred_element_type=jnp.float32)
             + jnp.dot(c, wm_ref[pl.ds(D, D), :],
                       preferred_element_type=jnp.float32) + bm_ref[...])
        o_ref[...] = ln(m, gm_ref[...], bem_ref[...])

    def full(shape):
        return pl.BlockSpec(shape, lambda i, _r=len(shape): (0,) * _r)

    return pl.pallas_call(
        body,
        grid=(NBLK,),
        in_specs=[
            pl.BlockSpec((1, RB, D), lambda i: (0, i, 0)),
            pl.BlockSpec((1, RB, D), lambda i: (1, i, 0)),
            pl.BlockSpec((2, 1, DB, 16), lambda i: (0, 0, i, 0)),
            full((D, D)), full((1, D)),
            full((D, D)), full((1, D)),
            full((D, D)), full((1, D)), full((1, D)), full((1, D)),
            full((D, D)), full((1, D)), full((1, D)), full((1, D)),
            full((2 * D, D)), full((1, D)), full((1, D)), full((1, D)),
        ],
        out_specs=pl.BlockSpec((RB, D), lambda i: (i, 0)),
        out_shape=jax.ShapeDtypeStruct((N, D), jnp.float32),
    )(agg2, agg2, parts, W_row, b_row.reshape(1, D), W_col,
      b_col.reshape(1, D),
      W_rs, b_rs.reshape(1, D), g_rs.reshape(1, D), be_rs.reshape(1, D),
      W_cs, b_cs.reshape(1, D), g_cs.reshape(1, D), be_cs.reshape(1, D),
      W_m, b_m.reshape(1, D), g_m.reshape(1, D), be_m.reshape(1, D))


def _interleave_edges(graph):
    pad = EPAD - E
    padv = jnp.full((2, pad), N, jnp.int32)
    sd = jnp.concatenate([graph, padv], axis=1)
    sd = sd.reshape(2, NS, CH, CW)
    return jnp.transpose(sd, (1, 2, 0, 3))


def kernel(table_feat, row_graph, col_graph, W_row, b_row, W_col, b_col,
           W_rs, b_rs, g_rs, be_rs, W_cs, b_cs, g_cs, be_cs,
           W_m, b_m, g_m, be_m):
    f32 = jnp.float32

    sd_r = _interleave_edges(row_graph)
    sd_c = _interleave_edges(col_graph)

    NT = NBLK * RB
    degs = jnp.stack([
        jnp.bincount(row_graph[0], length=NT),
        jnp.bincount(col_graph[0], length=NT)]).astype(f32)
    degd = jnp.stack([
        jnp.bincount(row_graph[1], length=NT),
        jnp.bincount(col_graph[1], length=NT)]).astype(f32)
    degs = degs.reshape(NC, NBLK, DB, 16)
    degd = degd.reshape(NC, 1, NBLK * DB, 16)

    hrow, hcol = _scale_tc(table_feat, degs)

    hrow, hcol, degd = lax.optimization_barrier((hrow, hcol, degd))

    zeros = jnp.zeros((ZR, D), f32)
    agg2 = _spmm_sc(hrow, hcol, sd_r, sd_c, zeros)

    return _dense_tc(agg2, degd, W_row, b_row, W_col, b_col,
                     W_rs, b_rs, g_rs, be_rs, W_cs, b_cs, g_cs, be_cs,
                     W_m, b_m, g_m, be_m)


# speedup vs baseline: 1.0945x; 1.0945x over previous
import functools

import jax
import jax.numpy as jnp
from jax import lax
from jax.experimental import pallas as pl
from jax.experimental.pallas import tpu as pltpu
from jax.experimental.pallas import tpu_sc as plsc

N = 10000
E = 320000
D = 128

NS = 16
NC = 2
CW = 80
CH = 256
BS = 8
NB = CH // BS
KB = 4
EPAD = NS * CH * CW
NPAD = 10112
ZR = NPAD // NS

DEGR = 640

RB = 2048
NBLK = 5
DB = RB // 16


def _spmm_sc(h_row, h_col, sd_r, sd_c, zeros):
    mesh = plsc.VectorSubcoreMesh(core_axis_name="c", subcore_axis_name="s")

    @functools.partial(
        pl.kernel, mesh=mesh,
        out_type=jax.ShapeDtypeStruct((NC, NPAD, D), jnp.float32),
        scratch_types=(
            [pltpu.VMEM((BS, 2, CW), jnp.int32) for _ in range(4)]
            + [pltpu.VMEM((CW, D), jnp.float32) for _ in range(KB)]
            + [pltpu.VMEM_SHARED((NPAD, D), jnp.float32)]
            + [pltpu.SemaphoreType.DMA for _ in range(4 + 2 * KB)]
        ),
    )
    def k(hr, hc, sdr, sdc, z, out, *refs):
        ibs = refs[0:4]
        rows = refs[4:4 + KB]
        agg_sh = refs[4 + KB]
        semis = refs[5 + KB:9 + KB]
        semg = refs[9 + KB:9 + 2 * KB]
        sems = refs[9 + 2 * KB:9 + 3 * KB]
        cid = lax.axis_index("c")
        sid = lax.axis_index("s")

        pltpu.sync_copy(z, agg_sh.at[pl.ds(sid * ZR, ZR)])
        plsc.subcore_barrier()

        def run(h_hbm, sd_hbm):
            def idx_issue(b, cur):
                pltpu.async_copy(
                    sd_hbm.at[sid, pl.ds(b * BS, BS)], ibs[cur], semis[cur])

            def idx_wait(cur):
                pltpu.make_async_copy(
                    sd_hbm.at[sid, pl.ds(0, BS)], ibs[cur], semis[cur]).wait()

            def gather_issue(idx_ref, p):
                pltpu.async_copy(h_hbm.at[idx_ref], rows[p], semg[p])

            def gather_wait(p):
                pltpu.make_async_copy(
                    h_hbm.at[ibs[0].at[0, 0]], rows[p], semg[p]).wait()

            def scatter_issue(idx_ref, p):
                pltpu.async_copy(rows[p], agg_sh.at[idx_ref], sems[p],
                                 add=True)

            def scatter_wait(p):
                pltpu.make_async_copy(
                    rows[p], agg_sh.at[ibs[0].at[0, 1]], sems[p]).wait()

            idx_issue(0, 0)
            idx_issue(1, 1)
            idx_issue(2, 2)
            idx_wait(0)
            gather_issue(ibs[0].at[0, 0], 0)
            gather_issue(ibs[0].at[1, 0], 1)

            def block(b, cur):
                ib_cur = ibs[cur]
                for kk in range(BS):
                    j = b * BS + kk
                    pc = kk % KB
                    pn = (kk + 2) % KB
                    pv = (kk + 3) % KB
                    if kk == 2:
                        @pl.when(b + 3 < NB)
                        def _():
                            idx_issue(b + 3, (cur + 3) % 4)
                    if kk == BS - 2:
                        @pl.when(b + 1 < NB)
                        def _():
                            idx_wait((cur + 1) % 4)
                    if kk < BS - 2:
                        nidx = ib_cur.at[kk + 2, 0]
                    else:
                        nidx = ibs[(cur + 1) % 4].at[kk - (BS - 2), 0]

                    @pl.when(j + 2 < CH)
                    def _():
                        gather_issue(nidx, pn)

                    gather_wait(pc)
                    @pl.when(j >= 1)
                    def _():
                        scatter_wait(pv)

                    scatter_issue(ib_cur.at[kk, 1], pc)

            def quad(q, carry):
                for i in range(4):
                    block(4 * q + i, i)
                return carry

            lax.fori_loop(0, NB // 4, quad, 0)
            scatter_wait((CH - 1) % KB)

        @pl.when(cid == 0)
        def _():
            run(hr, sdr)

        @pl.when(cid == 1)
        def _():
            run(hc, sdc)

        plsc.subcore_barrier()
        pltpu.sync_copy(agg_sh.at[pl.ds(sid * ZR, ZR)],
                        out.at[cid, pl.ds(sid * ZR, ZR)])

    return k(h_row, h_col, sd_r, sd_c, zeros)


def _row_scale(block2d, scales_db16):
    x3 = block2d.reshape(DB, 16, D)
    s3 = scales_db16.reshape(DB, 16, 1)
    return (x3 * s3).reshape(RB, D)


def _deg_scale(deg_block):
    return lax.rsqrt(jnp.maximum(deg_block, 1.0))


def _scale_tc(x, degs):
    def body(x_ref, d_ref, o_ref):
        o_ref[0] = _row_scale(x_ref[...], _deg_scale(d_ref[0, 0]))

    return pl.pallas_call(
        body,
        grid=(NC, NBLK),
        in_specs=[
            pl.BlockSpec((RB, D), lambda g, i: (i, 0)),
            pl.BlockSpec((1, 1, DB, 16), lambda g, i: (g, i, 0, 0)),
        ],
        out_specs=pl.BlockSpec((1, RB, D), lambda g, i: (g, i, 0)),
        out_shape=jax.ShapeDtypeStruct((NC, NBLK * RB, D), jnp.float32),
    )(x, degs)


def _dense_tc(agg2, parts, W_row, b_row, W_col, b_col,
              W_rs, b_rs, g_rs, be_rs, W_cs, b_cs, g_cs, be_cs,
              W_m, b_m, g_m, be_m):
    def ln(x, gamma, beta):
        mu = jnp.mean(x, axis=-1, keepdims=True)
        xc = x - mu
        var = jnp.mean(xc * xc, axis=-1, keepdims=True)
        return xc * lax.rsqrt(var + 1e-5) * gamma + beta

    def body(ar_ref, ac_ref, p_ref, wr_ref, br_ref, wc_ref, bc_ref,
             wrs_ref, brs_ref, grs_ref, bers_ref,
             wcs_ref, bcs_ref, gcs_ref, becs_ref,
             wm_ref, bm_ref, gm_ref, bem_ref, o_ref):
        ar = _row_scale(ar_ref[0], _deg_scale(p_ref[0, 0]))
        ac = _row_scale(ac_ref[0], _deg_scale(p_ref[1, 0]))
        hr = jax.nn.relu(
            jnp.dot(ar, wr_ref[...],
                    preferred_element_type=jnp.float32) + br_ref[...])
        hc = jax.nn.relu(
            jnp.dot(ac, wc_ref[...],
                    preferred_element_type=jnp.float32) + bc_ref[...])
        r = ln(jnp.dot(hr, wrs_ref[...],
                       preferred_element_type=jnp.float32) + brs_ref[...],
               grs_ref[...], bers_ref[...])
        c = ln(jnp.dot(hc, wcs_ref[...],
                       preferred_element_type=jnp.float32) + bcs_ref[...],
               gcs_ref[...], becs_ref[...])
        m = (jnp.dot(r, wm_ref[pl.ds(0, D), :],
                     preferred_element_type=jnp.float32)
             + jnp.dot(c, wm_ref[pl.ds(D, D), :],
                       preferred_element_type=jnp.float32) + bm_ref[...])
        o_ref[...] = ln(m, gm_ref[...], bem_ref[...])

    def full(shape):
        return pl.BlockSpec(shape, lambda i, _r=len(shape): (0,) * _r)

    return pl.pallas_call(
        body,
        grid=(NBLK,),
        in_specs=[
            pl.BlockSpec((1, RB, D), lambda i: (0, i, 0)),
            pl.BlockSpec((1, RB, D), lambda i: (1, i, 0)),
            pl.BlockSpec((2, 1, DB, 16), lambda i: (0, 0, i, 0)),
            full((D, D)), full((1, D)),
            full((D, D)), full((1, D)),
            full((D, D)), full((1, D)), full((1, D)), full((1, D)),
            full((D, D)), full((1, D)), full((1, D)), full((1, D)),
            full((2 * D, D)), full((1, D)), full((1, D)), full((1, D)),
        ],
        out_specs=pl.BlockSpec((RB, D), lambda i: (i, 0)),
        out_shape=jax.ShapeDtypeStruct((NBLK * RB, D), jnp.float32),
    )(agg2, agg2, parts, W_row, b_row.reshape(1, D), W_col,
      b_col.reshape(1, D),
      W_rs, b_rs.reshape(1, D), g_rs.reshape(1, D), be_rs.reshape(1, D),
      W_cs, b_cs.reshape(1, D), g_cs.reshape(1, D), be_cs.reshape(1, D),
      W_m, b_m.reshape(1, D), g_m.reshape(1, D), be_m.reshape(1, D))


def _interleave_edges(graph):
    pad = EPAD - E
    padv = jnp.full((2, pad), N, jnp.int32)
    sd = jnp.concatenate([graph, padv], axis=1)
    sd = sd.reshape(2, NS, CH, CW)
    return jnp.transpose(sd, (1, 2, 0, 3))


def kernel(table_feat, row_graph, col_graph, W_row, b_row, W_col, b_col,
           W_rs, b_rs, g_rs, be_rs, W_cs, b_cs, g_cs, be_cs,
           W_m, b_m, g_m, be_m):
    f32 = jnp.float32

    sd_r = _interleave_edges(row_graph)
    sd_c = _interleave_edges(col_graph)

    NT = NBLK * RB
    degs = jnp.stack([
        jnp.bincount(row_graph[0], length=NT),
        jnp.bincount(col_graph[0], length=NT)]).astype(f32)
    degd = jnp.stack([
        jnp.bincount(row_graph[1], length=NT),
        jnp.bincount(col_graph[1], length=NT)]).astype(f32)
    degs = degs.reshape(NC, NBLK, DB, 16)
    degd = degd.reshape(NC, 1, NBLK * DB, 16)

    xpad = jnp.concatenate(
        [table_feat, jnp.zeros((NT - N, D), f32)], axis=0)
    h2 = _scale_tc(xpad, degs)

    h2, degd = lax.optimization_barrier((h2, degd))

    zeros = jnp.zeros((ZR, D), f32)
    agg2 = _spmm_sc(h2[0], h2[1], sd_r, sd_c, zeros)

    agg2p = jnp.concatenate(
        [agg2, jnp.zeros((NC, NBLK * RB - NPAD, D), f32)], axis=1)

    out = _dense_tc(agg2p, degd, W_row, b_row, W_col, b_col,
                    W_rs, b_rs, g_rs, be_rs, W_cs, b_cs, g_cs, be_cs,
                    W_m, b_m, g_m, be_m)
    return out[:N]
